# baseline (speedup 1.0000x reference)
import functools

import jax
import jax.numpy as jnp
from jax import lax
from jax.experimental import pallas as pl
from jax.experimental.pallas import tpu as pltpu
from jax.experimental.pallas import tpu_sc as plsc


_ROWS_PER_STEP = 8
_SC_ROWS_PER_WORKER = 8


def _sc_gather(W, idx):
    n = idx.shape[0]
    e = W.shape[1]
    rpw = _SC_ROWS_PER_WORKER
    nw = n // rpw
    mesh = plsc.VectorSubcoreMesh(core_axis_name="c", subcore_axis_name="s")

    @functools.partial(
        pl.kernel,
        mesh=mesh,
        out_type=jax.ShapeDtypeStruct((n, e), jnp.float32),
        scratch_types=[
            pltpu.VMEM((rpw,), jnp.int32),
            pltpu.VMEM((rpw, e), jnp.float32),
            pltpu.SemaphoreType.DMA,
        ],
    )
    def gather_kernel(table_hbm, idx_hbm, out_hbm, idx_v, rows_v, sem):
        wid = lax.axis_index("s") * 2 + lax.axis_index("c")

        @pl.when(wid < nw)
        def _():
            base = wid * rpw
            pltpu.sync_copy(idx_hbm.at[pl.ds(base, rpw)], idx_v)
            pltpu.async_copy(table_hbm.at[idx_v], rows_v, sem).wait()
            pltpu.sync_copy(rows_v, out_hbm.at[pl.ds(base, rpw)])

    return gather_kernel(W, idx)


def _add_body(x_ref, emb_ref, o_ref):
    o_ref[...] = (x_ref[...] + emb_ref[...][:, None, :]) * 0.5


def kernel(x, blocks, W):
    B, NB, T, E = x.shape
    R = _ROWS_PER_STEP
    xf = x.reshape(B * NB, T, E)
    idx = blocks.reshape(-1).astype(jnp.int32)

    emb = _sc_gather(W, idx)

    out = pl.pallas_call(
        _add_body,
        grid=(B * NB // R,),
        in_specs=[
            pl.BlockSpec((R, T, E), lambda i: (i, 0, 0)),
            pl.BlockSpec((R, E), lambda i: (i, 0)),
        ],
        out_specs=pl.BlockSpec((R, T, E), lambda i: (i, 0, 0)),
        out_shape=jax.ShapeDtypeStruct((B * NB, T, E), x.dtype),
    )(xf, emb)
    return out.reshape(B, NB * T, E)


# speedup vs baseline: 1.0145x; 1.0145x over previous
import functools

import jax
import jax.numpy as jnp
from jax import lax
from jax.experimental import pallas as pl
from jax.experimental.pallas import tpu as pltpu
from jax.experimental.pallas import tpu_sc as plsc


_ROWS_PER_STEP = 8
_SC_ROWS_PER_WORKER = 8


def _sc_gather(W, idx):
    n = idx.shape[0]
    e = W.shape[1]
    rpw = _SC_ROWS_PER_WORKER
    nw = n // rpw
    mesh = plsc.VectorSubcoreMesh(
        core_axis_name="c", subcore_axis_name="s", num_cores=1
    )

    @functools.partial(
        pl.kernel,
        mesh=mesh,
        out_type=jax.ShapeDtypeStruct((n, e), jnp.float32),
        scratch_types=[
            pltpu.VMEM((rpw,), jnp.int32),
            pltpu.VMEM((rpw, e), jnp.float32),
            pltpu.SemaphoreType.DMA,
        ],
    )
    def gather_kernel(table_hbm, idx_hbm, out_hbm, idx_v, rows_v, sem):
        wid = lax.axis_index("s") + 16 * lax.axis_index("c")

        @pl.when(wid < nw)
        def _():
            base = wid * rpw
            pltpu.sync_copy(idx_hbm.at[pl.ds(base, rpw)], idx_v)
            pltpu.async_copy(table_hbm.at[idx_v], rows_v, sem).wait()
            pltpu.sync_copy(rows_v, out_hbm.at[pl.ds(base, rpw)])

    return gather_kernel(W, idx)


def _add_body(x_ref, emb_ref, o_ref):
    o_ref[...] = (x_ref[...] + emb_ref[...][:, None, :]) * 0.5


def kernel(x, blocks, W):
    B, NB, T, E = x.shape
    R = _ROWS_PER_STEP
    xf = x.reshape(B * NB, T, E)
    idx = blocks.reshape(-1).astype(jnp.int32)

    emb = _sc_gather(W, idx)

    out = pl.pallas_call(
        _add_body,
        grid=(B * NB // R,),
        in_specs=[
            pl.BlockSpec((R, T, E), lambda i: (i, 0, 0)),
            pl.BlockSpec((R, E), lambda i: (i, 0)),
        ],
        out_specs=pl.BlockSpec((R, T, E), lambda i: (i, 0, 0)),
        out_shape=jax.ShapeDtypeStruct((B * NB, T, E), x.dtype),
    )(xf, emb)
    return out.reshape(B, NB * T, E)


# speedup vs baseline: 1.0159x; 1.0015x over previous
import functools

import jax
import jax.numpy as jnp
from jax import lax
from jax.experimental import pallas as pl
from jax.experimental.pallas import tpu as pltpu
from jax.experimental.pallas import tpu_sc as plsc


_ROWS_PER_STEP = 8
_SC_ROWS_PER_WORKER = 8


def _sc_gather(W, idx):
    n = idx.shape[0]
    e = W.shape[1]
    rpw = _SC_ROWS_PER_WORKER
    nw = n // rpw
    mesh = plsc.VectorSubcoreMesh(
        core_axis_name="c", subcore_axis_name="s", num_cores=1
    )

    @functools.partial(
        pl.kernel,
        mesh=mesh,
        out_type=jax.ShapeDtypeStruct((n, e), jnp.float32),
        scratch_types=[
            pltpu.VMEM((rpw,), jnp.int32),
            pltpu.VMEM((rpw, e), jnp.float32),
            pltpu.SemaphoreType.DMA,
        ],
    )
    def gather_kernel(table_hbm, idx_hbm, out_hbm, idx_v, rows_v, sem):
        wid = lax.axis_index("s") + 16 * lax.axis_index("c")

        def work():
            base = wid * rpw
            pltpu.sync_copy(idx_hbm.at[pl.ds(base, rpw)], idx_v)
            pltpu.async_copy(table_hbm.at[idx_v], rows_v, sem).wait()
            pltpu.sync_copy(rows_v, out_hbm.at[pl.ds(base, rpw)])

        if nw < 16:
            pl.when(wid < nw)(work)
        else:
            work()

    return gather_kernel(W, idx)


def _add_body(x_ref, emb_ref, o_ref):
    o_ref[...] = (x_ref[...] + emb_ref[...][:, None, :]) * 0.5


def kernel(x, blocks, W):
    B, NB, T, E = x.shape
    R = _ROWS_PER_STEP
    xf = x.reshape(B * NB, T, E)
    idx = blocks.reshape(-1).astype(jnp.int32)

    emb = _sc_gather(W, idx)

    out = pl.pallas_call(
        _add_body,
        grid=(B * NB // R,),
        in_specs=[
            pl.BlockSpec((R, T, E), lambda i: (i, 0, 0)),
            pl.BlockSpec((R, E), lambda i: (i, 0)),
        ],
        out_specs=pl.BlockSpec((R, T, E), lambda i: (i, 0, 0)),
        out_shape=jax.ShapeDtypeStruct((B * NB, T, E), x.dtype),
    )(xf, emb)
    return out.reshape(B, NB * T, E)
